# Initial kernel scaffold; baseline (speedup 1.0000x reference)
import jax
import jax.numpy as jnp
from jax.experimental import pallas as pl


def kernel(x, edge_index0, edge_index1, W0, b0, W1, b1):
    raise NotImplementedError("write your pallas kernel here")


# speedup vs baseline: 8.6030x; 8.6030x over previous
import functools

import jax
import jax.numpy as jnp
from jax import lax
from jax.experimental import pallas as pl
from jax.experimental.pallas import tpu as pltpu
from jax.experimental.pallas import tpu_sc as plsc

N = 10000
E = 320000
D_IN = 128
D_H = 256
N_CLS = 64

NC = 2
NS = 16
NW = NC * NS

PAD = 16
C = 80
EPW = E // NW
K = EPW // C

ROWS_A = 640
ROWS_B = N - ROWS_A * (NS - 1)


def _make_agg(Dp):
    mesh = plsc.VectorSubcoreMesh(core_axis_name="c", subcore_axis_name="s",
                                  num_cores=NC, num_subcores=NS)

    @functools.partial(
        pl.kernel,
        out_type=jax.ShapeDtypeStruct((NC, N, Dp), jnp.float32),
        mesh=mesh,
        scratch_types=[
            pltpu.VMEM_SHARED((N, Dp), jnp.float32),
            pltpu.VMEM((K, C), jnp.int32),
            pltpu.VMEM((K, C), jnp.int32),
            pltpu.VMEM((C, Dp), jnp.float32),
            pltpu.VMEM((16, Dp), jnp.float32),
            pltpu.SemaphoreType.DMA,
        ],
        compiler_params=pltpu.CompilerParams(use_tc_tiling_on_sc=False),
    )
    def agg(feat, src2, dst2, sums_out, acc, src_v, dst_v, rows, zrow, sem):
        cid = lax.axis_index("c")
        sid = lax.axis_index("s")
        wid = cid * NS + sid

        z16 = jnp.zeros((16,), jnp.float32)

        def zfill(i, _):
            r = i // (Dp // 16)
            col = (i % (Dp // 16)) * 16
            zrow[r, pl.ds(col, 16)] = z16
            return 0

        lax.fori_loop(0, 16 * (Dp // 16), zfill, 0)

        @pl.when(sid < NS - 1)
        def _():
            def zb(k, _):
                pltpu.sync_copy(zrow, acc.at[pl.ds(sid * ROWS_A + k * 16, 16)])
                return 0
            lax.fori_loop(0, ROWS_A // 16, zb, 0)

        @pl.when(sid == NS - 1)
        def _():
            def zb(k, _):
                pltpu.sync_copy(zrow, acc.at[pl.ds((NS - 1) * ROWS_A + k * 16, 16)])
                return 0
            lax.fori_loop(0, ROWS_B // 16, zb, 0)

        pltpu.sync_copy(src2.at[wid], src_v)
        pltpu.sync_copy(dst2.at[wid], dst_v)

        plsc.subcore_barrier()

        def ebody(j, _):
            pltpu.async_copy(feat.at[src_v.at[j]], rows, sem).wait()
            pltpu.sync_copy(rows, acc.at[dst_v.at[j]], add=True)
            return 0

        lax.fori_loop(0, K, ebody, 0)

        plsc.subcore_barrier()

        @pl.when(sid < NS - 1)
        def _():
            base = sid * ROWS_A
            pltpu.sync_copy(acc.at[pl.ds(base, ROWS_A)],
                            sums_out.at[cid, pl.ds(base, ROWS_A)])

        @pl.when(sid == NS - 1)
        def _():
            base = (NS - 1) * ROWS_A
            pltpu.sync_copy(acc.at[pl.ds(base, ROWS_B)],
                            sums_out.at[cid, pl.ds(base, ROWS_B)])

    return agg


_agg_l0 = _make_agg(D_IN + PAD)
_agg_l1 = _make_agg(N_CLS + PAD)

_R1 = 1000
_R2 = 1000


def _tc_fuse(p0, W0, b0, W1):
    D0 = D_IN + PAD

    def body(p_ref, w0_ref, b0_ref, w1_ref, z_ref):
        a = p_ref[0] + p_ref[1]
        c = a[:, D_IN:D_IN + 1]
        inv = 1.0 / jnp.maximum(c, 1.0)
        mask = (c > 0.0).astype(jnp.float32)
        feats = a[:, :D_IN] * inv
        h = jnp.dot(feats, w0_ref[...], preferred_element_type=jnp.float32)
        h = jnp.maximum(h + b0_ref[...] * mask, 0.0)
        z = jnp.dot(h, w1_ref[...], preferred_element_type=jnp.float32)
        z_ref[...] = jnp.concatenate(
            [z, jnp.ones((_R1, 1), jnp.float32),
             jnp.zeros((_R1, PAD - 1), jnp.float32)], axis=1)

    return pl.pallas_call(
        body,
        grid=(N // _R1,),
        in_specs=[
            pl.BlockSpec((NC, _R1, D0), lambda i: (0, i, 0)),
            pl.BlockSpec((D_IN, D_H), lambda i: (0, 0)),
            pl.BlockSpec((1, D_H), lambda i: (0, 0)),
            pl.BlockSpec((D_H, N_CLS), lambda i: (0, 0)),
        ],
        out_specs=pl.BlockSpec((_R1, N_CLS + PAD), lambda i: (i, 0)),
        out_shape=jax.ShapeDtypeStruct((N, N_CLS + PAD), jnp.float32),
    )(p0, W0, b0, W1)


def _tc_final(p1, b1):
    D1 = N_CLS + PAD

    def body(p_ref, b_ref, o_ref):
        s = p_ref[0] + p_ref[1]
        c = s[:, N_CLS:N_CLS + 1]
        inv = 1.0 / jnp.maximum(c, 1.0)
        mask = (c > 0.0).astype(jnp.float32)
        o_ref[...] = s[:, :N_CLS] * inv + b_ref[...] * mask

    return pl.pallas_call(
        body,
        grid=(N // _R2,),
        in_specs=[
            pl.BlockSpec((NC, _R2, D1), lambda i: (0, i, 0)),
            pl.BlockSpec((1, N_CLS), lambda i: (0, 0)),
        ],
        out_specs=pl.BlockSpec((_R2, N_CLS), lambda i: (i, 0)),
        out_shape=jax.ShapeDtypeStruct((N, N_CLS), jnp.float32),
    )(p1, b1)


def kernel(x, edge_index0, edge_index1, W0, b0, W1, b1):
    x_pad = jnp.concatenate(
        [x, jnp.ones((N, 1), jnp.float32),
         jnp.zeros((N, PAD - 1), jnp.float32)], axis=1)
    src0 = edge_index0[0].reshape(NW, K, C)
    dst0 = edge_index0[1].reshape(NW, K, C)
    src1 = edge_index1[0].reshape(NW, K, C)
    dst1 = edge_index1[1].reshape(NW, K, C)

    p0 = _agg_l0(x_pad, src0, dst0)
    z_pad = _tc_fuse(p0, W0, b0.reshape(1, D_H), W1)
    p1 = _agg_l1(z_pad, src1, dst1)
    return _tc_final(p1, b1.reshape(1, N_CLS))


# speedup vs baseline: 11.7299x; 1.3635x over previous
import functools

import jax
import jax.numpy as jnp
from jax import lax
from jax.experimental import pallas as pl
from jax.experimental.pallas import tpu as pltpu
from jax.experimental.pallas import tpu_sc as plsc

N = 10000
E = 320000
D_IN = 128
D_H = 256
N_CLS = 64

NC = 2
NS = 16
NW = NC * NS

PAD = 16
C = 125
EPW = E // NW
K = EPW // C

ROWS_A = 640
ROWS_B = N - ROWS_A * (NS - 1)
ZR = 16


def _make_agg(Dp):
    mesh = plsc.VectorSubcoreMesh(core_axis_name="c", subcore_axis_name="s",
                                  num_cores=NC, num_subcores=NS)

    @functools.partial(
        pl.kernel,
        out_type=jax.ShapeDtypeStruct((NC, N, Dp), jnp.float32),
        mesh=mesh,
        scratch_types=[
            pltpu.VMEM_SHARED((N, Dp), jnp.float32),
            pltpu.VMEM((2, C), jnp.int32),
            pltpu.VMEM((2, C), jnp.int32),
            pltpu.VMEM((C, Dp), jnp.float32),
            pltpu.VMEM((C, Dp), jnp.float32),
            pltpu.VMEM((ZR, Dp), jnp.float32),
            pltpu.SemaphoreType.DMA,
            pltpu.SemaphoreType.DMA,
            pltpu.SemaphoreType.DMA,
            pltpu.SemaphoreType.DMA,
        ],
        compiler_params=pltpu.CompilerParams(use_tc_tiling_on_sc=False),
    )
    def agg(feat, eidx, sums_out, acc, idx_a, idx_b,
            rows_a, rows_b, zrow, sem_a, sem_b, sem_ia, sem_ib):
        cid = lax.axis_index("c")
        sid = lax.axis_index("s")
        wid = cid * NS + sid

        z16 = jnp.zeros((16,), jnp.float32)

        def zfill(i, _):
            r = i // (Dp // 16)
            col = (i % (Dp // 16)) * 16
            zrow[r, pl.ds(col, 16)] = z16
            return 0

        lax.fori_loop(0, ZR * (Dp // 16), zfill, 0)

        @pl.when(sid < NS - 1)
        def _():
            def zb(k, _):
                pltpu.sync_copy(zrow, acc.at[pl.ds(sid * ROWS_A + k * ZR, ZR)])
                return 0
            lax.fori_loop(0, ROWS_A // ZR, zb, 0)

        @pl.when(sid == NS - 1)
        def _():
            def zb(k, _):
                pltpu.sync_copy(zrow, acc.at[pl.ds((NS - 1) * ROWS_A + k * ZR, ZR)])
                return 0
            lax.fori_loop(0, ROWS_B // ZR, zb, 0)

        plsc.subcore_barrier()

        pltpu.sync_copy(eidx.at[wid, 0], idx_a)
        pltpu.async_copy(feat.at[idx_a.at[0]], rows_a, sem_a)
        pltpu.async_copy(eidx.at[wid, 1], idx_b, sem_ib)

        def ebody(i, _):
            j = 2 * i
            pltpu.make_async_copy(feat.at[idx_a.at[0]], rows_a, sem_a).wait()
            pltpu.make_async_copy(eidx.at[wid, j + 1], idx_b, sem_ib).wait()
            pltpu.async_copy(feat.at[idx_b.at[0]], rows_b, sem_b)
            pltpu.sync_copy(rows_a, acc.at[idx_a.at[1]], add=True)

            @pl.when(j + 2 < K)
            def _():
                pltpu.async_copy(eidx.at[wid, j + 2], idx_a, sem_ia)

            pltpu.make_async_copy(feat.at[idx_b.at[0]], rows_b, sem_b).wait()

            @pl.when(j + 2 < K)
            def _():
                pltpu.make_async_copy(eidx.at[wid, j + 2], idx_a, sem_ia).wait()
                pltpu.async_copy(feat.at[idx_a.at[0]], rows_a, sem_a)

            pltpu.sync_copy(rows_b, acc.at[idx_b.at[1]], add=True)

            @pl.when(j + 3 < K)
            def _():
                pltpu.async_copy(eidx.at[wid, j + 3], idx_b, sem_ib)
            return 0

        lax.fori_loop(0, K // 2, ebody, 0)

        plsc.subcore_barrier()

        @pl.when(sid < NS - 1)
        def _():
            base = sid * ROWS_A
            pltpu.sync_copy(acc.at[pl.ds(base, ROWS_A)],
                            sums_out.at[cid, pl.ds(base, ROWS_A)])

        @pl.when(sid == NS - 1)
        def _():
            base = (NS - 1) * ROWS_A
            pltpu.sync_copy(acc.at[pl.ds(base, ROWS_B)],
                            sums_out.at[cid, pl.ds(base, ROWS_B)])

    return agg


_agg_l0 = _make_agg(D_IN + PAD)
_agg_l1 = _make_agg(N_CLS + PAD)

_R1 = 1000
_R2 = 1000


def _tc_fuse(p0, W0, b0, W1):
    D0 = D_IN + PAD

    def body(p_ref, w0_ref, b0_ref, w1_ref, z_ref):
        a = p_ref[0] + p_ref[1]
        c = a[:, D_IN:D_IN + 1]
        inv = 1.0 / jnp.maximum(c, 1.0)
        mask = (c > 0.0).astype(jnp.float32)
        feats = a[:, :D_IN] * inv
        h = jnp.dot(feats, w0_ref[...], preferred_element_type=jnp.float32)
        h = jnp.maximum(h + b0_ref[...] * mask, 0.0)
        z = jnp.dot(h, w1_ref[...], preferred_element_type=jnp.float32)
        z_ref[...] = jnp.concatenate(
            [z, jnp.ones((_R1, 1), jnp.float32),
             jnp.zeros((_R1, PAD - 1), jnp.float32)], axis=1)

    return pl.pallas_call(
        body,
        grid=(N // _R1,),
        in_specs=[
            pl.BlockSpec((NC, _R1, D0), lambda i: (0, i, 0)),
            pl.BlockSpec((D_IN, D_H), lambda i: (0, 0)),
            pl.BlockSpec((1, D_H), lambda i: (0, 0)),
            pl.BlockSpec((D_H, N_CLS), lambda i: (0, 0)),
        ],
        out_specs=pl.BlockSpec((_R1, N_CLS + PAD), lambda i: (i, 0)),
        out_shape=jax.ShapeDtypeStruct((N, N_CLS + PAD), jnp.float32),
    )(p0, W0, b0, W1)


def _tc_final(p1, b1):
    D1 = N_CLS + PAD

    def body(p_ref, b_ref, o_ref):
        s = p_ref[0] + p_ref[1]
        c = s[:, N_CLS:N_CLS + 1]
        inv = 1.0 / jnp.maximum(c, 1.0)
        mask = (c > 0.0).astype(jnp.float32)
        o_ref[...] = s[:, :N_CLS] * inv + b_ref[...] * mask

    return pl.pallas_call(
        body,
        grid=(N // _R2,),
        in_specs=[
            pl.BlockSpec((NC, _R2, D1), lambda i: (0, i, 0)),
            pl.BlockSpec((1, N_CLS), lambda i: (0, 0)),
        ],
        out_specs=pl.BlockSpec((_R2, N_CLS), lambda i: (i, 0)),
        out_shape=jax.ShapeDtypeStruct((N, N_CLS), jnp.float32),
    )(p1, b1)


def kernel(x, edge_index0, edge_index1, W0, b0, W1, b1):
    x_pad = jnp.concatenate(
        [x, jnp.ones((N, 1), jnp.float32),
         jnp.zeros((N, PAD - 1), jnp.float32)], axis=1)
    eidx0 = jnp.stack([edge_index0[0].reshape(NW, K, C),
                       edge_index0[1].reshape(NW, K, C)], axis=2)
    eidx1 = jnp.stack([edge_index1[0].reshape(NW, K, C),
                       edge_index1[1].reshape(NW, K, C)], axis=2)

    p0 = _agg_l0(x_pad, eidx0)
    z_pad = _tc_fuse(p0, W0, b0.reshape(1, D_H), W1)
    p1 = _agg_l1(z_pad, eidx1)
    return _tc_final(p1, b1.reshape(1, N_CLS))
